# baseline (speedup 1.0000x reference)
import functools

import jax
import jax.numpy as jnp
from jax.experimental import pallas as pl

N = 4096
BM = 512


def _ext_width(d):
    return 128 * ((d + 1 + 127) // 128)


def _proj(h, W, a1, a2, whe_ref, f1_ref, f2_ref):
    d = W.shape[1]
    wh = jnp.dot(h, W, preferred_element_type=jnp.float32)
    whe_ref[:, :d] = wh.astype(jnp.bfloat16)
    whe_ref[:, d:] = jnp.ones_like(whe_ref[:, d:])
    f1_ref[:] = jnp.dot(wh, a1, preferred_element_type=jnp.float32)
    f2_ref[:] = jnp.dot(wh, a2, preferred_element_type=jnp.float32)


def _setup_kernel(adj_ref, x_ref, w_ref, a1_ref, a2_ref,
                  mask_ref, whe_ref, f1_ref, f2_ref):
    mask_ref[:] = (adj_ref[:] > 0).astype(jnp.float8_e4m3fn)
    _proj(x_ref[:], w_ref[:], a1_ref[:], a2_ref[:], whe_ref, f1_ref, f2_ref)


def _att_body(f1_ref, f2t_ref, mask_ref, whe_ref, d):
    log2e = jnp.float32(1.4426950408889634)
    f1 = f1_ref[:]
    f2 = f2t_ref[:]
    m2 = jnp.max(f2, axis=1, keepdims=True)
    fm = f1 + m2
    m = jnp.maximum(fm, 0.2 * fm)
    c1 = (f1 - m) * log2e
    c2 = (0.2 * f1 - m) * log2e
    g1 = f2 * log2e
    g2 = f2 * (0.2 * log2e)
    u1 = jnp.exp2(c1).astype(jnp.bfloat16)
    u2 = jnp.exp2(c2).astype(jnp.bfloat16)
    v1 = jnp.exp2(g1).astype(jnp.bfloat16)
    v2 = jnp.exp2(g2).astype(jnp.bfloat16)
    mb = mask_ref[:].astype(jnp.bfloat16)
    p = jnp.maximum(u1 * v1, u2 * v2) * mb
    acc = jnp.dot(p, whe_ref[:],
                  preferred_element_type=jnp.float32)
    s = acc[:, d:d + 1]
    r = acc[:, :d] / jnp.where(s > 0.0, s, 1.0)
    ones = jnp.ones((8, N), jnp.bfloat16)
    cm = jnp.dot(ones, whe_ref[:], preferred_element_type=jnp.float32)
    r = jnp.where(s > 0.0, r, cm[0:1, :d] * (1.0 / N))
    return jnp.where(r > 0, r, jnp.exp(jnp.minimum(r, 0.0)) - 1.0)


def _att_next_kernel(f1_ref, f2t_ref, mask_ref, whe_ref,
                     w_ref, a1_ref, a2_ref, *out_refs, d, emit_h):
    h = _att_body(f1_ref, f2t_ref, mask_ref, whe_ref, d)
    if emit_h:
        h_ref, whe2_ref, f12_ref, f22_ref = out_refs
        h_ref[:] = h
    else:
        whe2_ref, f12_ref, f22_ref = out_refs
    _proj(h, w_ref[:], a1_ref[:], a2_ref[:], whe2_ref, f12_ref, f22_ref)


def _att_last_kernel(f1_ref, f2t_ref, mask_ref, whe_ref, out_ref, *, d):
    out_ref[:] = _att_body(f1_ref, f2t_ref, mask_ref, whe_ref, d)


def _att_next(f1, f2, mask8, whe, W, a, d, emit_h=False):
    dn = W.shape[1]
    ew = _ext_width(dn)
    grid = (N // BM,)
    ewp = whe.shape[1]
    in_specs = [
        pl.BlockSpec((BM, 1), lambda i: (i, 0)),
        pl.BlockSpec((1, N), lambda i: (0, 0)),
        pl.BlockSpec((BM, N), lambda i: (i, 0)),
        pl.BlockSpec((N, ewp), lambda i: (0, 0)),
        pl.BlockSpec((W.shape[0], dn), lambda i: (0, 0)),
        pl.BlockSpec((dn, 1), lambda i: (0, 0)),
        pl.BlockSpec((dn, 1), lambda i: (0, 0)),
    ]
    out_specs = [
        pl.BlockSpec((BM, ew), lambda i: (i, 0)),
        pl.BlockSpec((BM, 1), lambda i: (i, 0)),
        pl.BlockSpec((BM, 1), lambda i: (i, 0)),
    ]
    out_shape = [
        jax.ShapeDtypeStruct((N, ew), jnp.bfloat16),
        jax.ShapeDtypeStruct((N, 1), jnp.float32),
        jax.ShapeDtypeStruct((N, 1), jnp.float32),
    ]
    if emit_h:
        out_specs = [pl.BlockSpec((BM, d), lambda i: (i, 0))] + out_specs
        out_shape = [jax.ShapeDtypeStruct((N, d), jnp.float32)] + out_shape
    return pl.pallas_call(
        functools.partial(_att_next_kernel, d=d, emit_h=emit_h),
        grid=grid,
        in_specs=in_specs,
        out_specs=out_specs,
        out_shape=out_shape,
    )(f1, f2.T, mask8, whe, W, a[:dn].reshape(dn, 1), a[dn:].reshape(dn, 1))


BS = 256


@jax.jit
def _run(x, adj, W_e0, a_e0, W_e1, a_e1, W_d0, a_d0, W_d1, a_d1):
    grid = (N // BM,)
    d0 = W_e0.shape[1]
    ew0 = _ext_width(d0)
    mask8, whe, f1, f2 = pl.pallas_call(
        _setup_kernel,
        grid=(N // BS,),
        in_specs=[
            pl.BlockSpec((BS, N), lambda i: (i, 0)),
            pl.BlockSpec((BS, x.shape[1]), lambda i: (i, 0)),
            pl.BlockSpec((x.shape[1], d0), lambda i: (0, 0)),
            pl.BlockSpec((d0, 1), lambda i: (0, 0)),
            pl.BlockSpec((d0, 1), lambda i: (0, 0)),
        ],
        out_specs=[
            pl.BlockSpec((BS, N), lambda i: (i, 0)),
            pl.BlockSpec((BS, ew0), lambda i: (i, 0)),
            pl.BlockSpec((BS, 1), lambda i: (i, 0)),
            pl.BlockSpec((BS, 1), lambda i: (i, 0)),
        ],
        out_shape=[
            jax.ShapeDtypeStruct((N, N), jnp.float8_e4m3fn),
            jax.ShapeDtypeStruct((N, ew0), jnp.bfloat16),
            jax.ShapeDtypeStruct((N, 1), jnp.float32),
            jax.ShapeDtypeStruct((N, 1), jnp.float32),
        ],
    )(adj, x, W_e0, a_e0[:d0].reshape(d0, 1), a_e0[d0:].reshape(d0, 1))

    whe, f1, f2 = _att_next(f1, f2, mask8, whe, W_e1, a_e1, d0)
    d1 = W_e1.shape[1]
    z, whe, f1, f2 = _att_next(f1, f2, mask8, whe, W_d0, a_d0, d1, emit_h=True)
    d2 = W_d0.shape[1]
    whe, f1, f2 = _att_next(f1, f2, mask8, whe, W_d1, a_d1, d2)
    d3 = W_d1.shape[1]
    x_hat = pl.pallas_call(
        functools.partial(_att_last_kernel, d=d3),
        grid=grid,
        in_specs=[
            pl.BlockSpec((BM, 1), lambda i: (i, 0)),
            pl.BlockSpec((1, N), lambda i: (0, 0)),
            pl.BlockSpec((BM, N), lambda i: (i, 0)),
            pl.BlockSpec((N, whe.shape[1]), lambda i: (0, 0)),
        ],
        out_specs=pl.BlockSpec((BM, d3), lambda i: (i, 0)),
        out_shape=jax.ShapeDtypeStruct((N, d3), jnp.float32),
    )(f1, f2.T, mask8, whe)
    return z, x_hat


def kernel(x, adj, W_e0, a_e0, W_e1, a_e1, W_d0, a_d0, W_d1, a_d1):
    return _run(x, adj, W_e0, a_e0, W_e1, a_e1, W_d0, a_d0, W_d1, a_d1)


# speedup vs baseline: 1.0593x; 1.0593x over previous
import functools

import jax
import jax.numpy as jnp
from jax.experimental import pallas as pl

N = 4096
BM = 1024


def _ext_width(d):
    return 128 * ((d + 1 + 127) // 128)


def _proj(h, W, a1, a2, whe_ref, f1_ref, f2_ref):
    d = W.shape[1]
    wh = jnp.dot(h.astype(jnp.bfloat16), W.astype(jnp.bfloat16),
                 preferred_element_type=jnp.float32)
    whe_ref[:, :d] = wh.astype(jnp.bfloat16)
    whe_ref[:, d:] = jnp.ones_like(whe_ref[:, d:])
    f1_ref[:] = jnp.dot(wh, a1, preferred_element_type=jnp.float32)
    f2_ref[:] = jnp.dot(wh, a2, preferred_element_type=jnp.float32)


def _setup_kernel(adj_ref, x_ref, w_ref, a1_ref, a2_ref,
                  mask_ref, whe_ref, f1_ref, f2_ref):
    mask_ref[:] = (adj_ref[:] > 0).astype(jnp.float8_e4m3fn)
    _proj(x_ref[:], w_ref[:], a1_ref[:], a2_ref[:], whe_ref, f1_ref, f2_ref)


def _att_body(f1_ref, f2t_ref, mask_ref, whe_ref, d):
    log2e = jnp.float32(1.4426950408889634)
    f1 = f1_ref[:]
    f2 = f2t_ref[:]
    m2 = jnp.max(f2, axis=1, keepdims=True)
    fm = f1 + m2
    m = jnp.maximum(fm, 0.2 * fm)
    c1 = (f1 - m) * log2e
    c2 = (0.2 * f1 - m) * log2e
    g1 = f2 * log2e
    g2 = f2 * (0.2 * log2e)
    u1 = jnp.exp2(c1).astype(jnp.bfloat16)
    u2 = jnp.exp2(c2).astype(jnp.bfloat16)
    v1 = jnp.exp2(g1).astype(jnp.bfloat16)
    v2 = jnp.exp2(g2).astype(jnp.bfloat16)
    mb = mask_ref[:].astype(jnp.bfloat16)
    p = jnp.maximum(u1 * v1, u2 * v2) * mb
    acc = jnp.dot(p, whe_ref[:],
                  preferred_element_type=jnp.float32)
    s = acc[:, d:d + 1]
    r = acc[:, :d] * (1.0 / jnp.where(s > 0.0, s, 1.0))
    ones = jnp.ones((8, N), jnp.bfloat16)
    cm = jnp.dot(ones, whe_ref[:], preferred_element_type=jnp.float32)
    r = jnp.where(s > 0.0, r, cm[0:1, :d] * (1.0 / N))
    return jnp.where(r > 0, r, jnp.exp(jnp.minimum(r, 0.0)) - 1.0)


def _att_next_kernel(f1_ref, f2t_ref, mask_ref, whe_ref,
                     w_ref, a1_ref, a2_ref, *out_refs, d, emit_h):
    h = _att_body(f1_ref, f2t_ref, mask_ref, whe_ref, d)
    if emit_h:
        h_ref, whe2_ref, f12_ref, f22_ref = out_refs
        h_ref[:] = h
    else:
        whe2_ref, f12_ref, f22_ref = out_refs
    _proj(h, w_ref[:], a1_ref[:], a2_ref[:], whe2_ref, f12_ref, f22_ref)


def _att_last_kernel(f1_ref, f2t_ref, mask_ref, whe_ref, out_ref, *, d):
    out_ref[:] = _att_body(f1_ref, f2t_ref, mask_ref, whe_ref, d)


def _att_next(f1, f2, mask8, whe, W, a, d, emit_h=False):
    dn = W.shape[1]
    ew = _ext_width(dn)
    grid = (N // BM,)
    ewp = whe.shape[1]
    in_specs = [
        pl.BlockSpec((BM, 1), lambda i: (i, 0)),
        pl.BlockSpec((1, N), lambda i: (0, 0)),
        pl.BlockSpec((BM, N), lambda i: (i, 0)),
        pl.BlockSpec((N, ewp), lambda i: (0, 0)),
        pl.BlockSpec((W.shape[0], dn), lambda i: (0, 0)),
        pl.BlockSpec((dn, 1), lambda i: (0, 0)),
        pl.BlockSpec((dn, 1), lambda i: (0, 0)),
    ]
    out_specs = [
        pl.BlockSpec((BM, ew), lambda i: (i, 0)),
        pl.BlockSpec((BM, 1), lambda i: (i, 0)),
        pl.BlockSpec((BM, 1), lambda i: (i, 0)),
    ]
    out_shape = [
        jax.ShapeDtypeStruct((N, ew), jnp.bfloat16),
        jax.ShapeDtypeStruct((N, 1), jnp.float32),
        jax.ShapeDtypeStruct((N, 1), jnp.float32),
    ]
    if emit_h:
        out_specs = [pl.BlockSpec((BM, d), lambda i: (i, 0))] + out_specs
        out_shape = [jax.ShapeDtypeStruct((N, d), jnp.float32)] + out_shape
    return pl.pallas_call(
        functools.partial(_att_next_kernel, d=d, emit_h=emit_h),
        grid=grid,
        in_specs=in_specs,
        out_specs=out_specs,
        out_shape=out_shape,
    )(f1, f2.T, mask8, whe, W, a[:dn].reshape(dn, 1), a[dn:].reshape(dn, 1))


BS = 256


@jax.jit
def _run(x, adj, W_e0, a_e0, W_e1, a_e1, W_d0, a_d0, W_d1, a_d1):
    grid = (N // BM,)
    d0 = W_e0.shape[1]
    ew0 = _ext_width(d0)
    mask8, whe, f1, f2 = pl.pallas_call(
        _setup_kernel,
        grid=(N // BS,),
        in_specs=[
            pl.BlockSpec((BS, N), lambda i: (i, 0)),
            pl.BlockSpec((BS, x.shape[1]), lambda i: (i, 0)),
            pl.BlockSpec((x.shape[1], d0), lambda i: (0, 0)),
            pl.BlockSpec((d0, 1), lambda i: (0, 0)),
            pl.BlockSpec((d0, 1), lambda i: (0, 0)),
        ],
        out_specs=[
            pl.BlockSpec((BS, N), lambda i: (i, 0)),
            pl.BlockSpec((BS, ew0), lambda i: (i, 0)),
            pl.BlockSpec((BS, 1), lambda i: (i, 0)),
            pl.BlockSpec((BS, 1), lambda i: (i, 0)),
        ],
        out_shape=[
            jax.ShapeDtypeStruct((N, N), jnp.float8_e4m3fn),
            jax.ShapeDtypeStruct((N, ew0), jnp.bfloat16),
            jax.ShapeDtypeStruct((N, 1), jnp.float32),
            jax.ShapeDtypeStruct((N, 1), jnp.float32),
        ],
    )(adj, x, W_e0, a_e0[:d0].reshape(d0, 1), a_e0[d0:].reshape(d0, 1))

    whe, f1, f2 = _att_next(f1, f2, mask8, whe, W_e1, a_e1, d0)
    d1 = W_e1.shape[1]
    z, whe, f1, f2 = _att_next(f1, f2, mask8, whe, W_d0, a_d0, d1, emit_h=True)
    d2 = W_d0.shape[1]
    whe, f1, f2 = _att_next(f1, f2, mask8, whe, W_d1, a_d1, d2)
    d3 = W_d1.shape[1]
    x_hat = pl.pallas_call(
        functools.partial(_att_last_kernel, d=d3),
        grid=grid,
        in_specs=[
            pl.BlockSpec((BM, 1), lambda i: (i, 0)),
            pl.BlockSpec((1, N), lambda i: (0, 0)),
            pl.BlockSpec((BM, N), lambda i: (i, 0)),
            pl.BlockSpec((N, whe.shape[1]), lambda i: (0, 0)),
        ],
        out_specs=pl.BlockSpec((BM, d3), lambda i: (i, 0)),
        out_shape=jax.ShapeDtypeStruct((N, d3), jnp.float32),
    )(f1, f2.T, mask8, whe)
    return z, x_hat


def kernel(x, adj, W_e0, a_e0, W_e1, a_e1, W_d0, a_d0, W_d1, a_d1):
    return _run(x, adj, W_e0, a_e0, W_e1, a_e1, W_d0, a_d0, W_d1, a_d1)


# speedup vs baseline: 1.1086x; 1.0465x over previous
import functools

import jax
import jax.numpy as jnp
from jax.experimental import pallas as pl

N = 4096
BM = 1024


def _ext_width(d):
    return 128 * ((d + 1 + 127) // 128)


def _proj(h, W, a1, a2, whe_ref, f1_ref, f2_ref):
    d = W.shape[1]
    wh = jnp.dot(h, W, preferred_element_type=jnp.float32)
    whe_ref[:, :d] = wh.astype(jnp.bfloat16)
    whe_ref[:, d:] = jnp.ones_like(whe_ref[:, d:])
    f1_ref[:] = jnp.dot(wh, a1, preferred_element_type=jnp.float32)
    f2_ref[:] = jnp.dot(wh, a2, preferred_element_type=jnp.float32)


def _setup_kernel(x_ref, w_ref, a1_ref, a2_ref, whe_ref, f1_ref, f2_ref):
    _proj(x_ref[:], w_ref[:], a1_ref[:], a2_ref[:], whe_ref, f1_ref, f2_ref)


def _att_body(f1_ref, f2t_ref, mb, whe_ref, d):
    log2e = jnp.float32(1.4426950408889634)
    f1 = f1_ref[:]
    f2 = f2t_ref[:]
    m2 = jnp.max(f2, axis=1, keepdims=True)
    fm = f1 + m2
    m = jnp.maximum(fm, 0.2 * fm)
    c1 = (f1 - m) * log2e
    c2 = (0.2 * f1 - m) * log2e
    g1 = f2 * log2e
    g2 = f2 * (0.2 * log2e)
    u1 = jnp.exp2(c1).astype(jnp.bfloat16)
    u2 = jnp.exp2(c2).astype(jnp.bfloat16)
    v1 = jnp.exp2(g1).astype(jnp.bfloat16)
    v2 = jnp.exp2(g2).astype(jnp.bfloat16)
    p = jnp.maximum(u1 * v1, u2 * v2) * mb
    acc = jnp.dot(p, whe_ref[:],
                  preferred_element_type=jnp.float32)
    s = acc[:, d:d + 1]
    r = acc[:, :d] * (1.0 / jnp.where(s > 0.0, s, 1.0))
    ones = jnp.ones((8, N), jnp.bfloat16)
    cm = jnp.dot(ones, whe_ref[:], preferred_element_type=jnp.float32)
    r = jnp.where(s > 0.0, r, cm[0:1, :d] * (1.0 / N))
    return jnp.where(r > 0, r, jnp.exp(jnp.minimum(r, 0.0)) - 1.0)


def _att_next_kernel(f1_ref, f2t_ref, mask_ref, whe_ref,
                     w_ref, a1_ref, a2_ref, *out_refs, d, emit_h):
    mb = mask_ref[:].astype(jnp.bfloat16)
    h = _att_body(f1_ref, f2t_ref, mb, whe_ref, d)
    if emit_h:
        h_ref, whe2_ref, f12_ref, f22_ref = out_refs
        h_ref[:] = h
    else:
        whe2_ref, f12_ref, f22_ref = out_refs
    _proj(h, w_ref[:], a1_ref[:], a2_ref[:], whe2_ref, f12_ref, f22_ref)


def _att_first_kernel(f1_ref, f2t_ref, adj_ref, whe_ref,
                      w_ref, a1_ref, a2_ref,
                      mask_ref, whe2_ref, f12_ref, f22_ref, *, d):
    pos = adj_ref[:] > 0
    mask_ref[:] = pos.astype(jnp.float8_e4m3fn)
    mb = pos.astype(jnp.bfloat16)
    h = _att_body(f1_ref, f2t_ref, mb, whe_ref, d)
    _proj(h, w_ref[:], a1_ref[:], a2_ref[:], whe2_ref, f12_ref, f22_ref)


def _att_last_kernel(f1_ref, f2t_ref, mask_ref, whe_ref, out_ref, *, d):
    mb = mask_ref[:].astype(jnp.bfloat16)
    out_ref[:] = _att_body(f1_ref, f2t_ref, mb, whe_ref, d)


def _att_next(f1, f2, mask8, whe, W, a, d, emit_h=False):
    dn = W.shape[1]
    ew = _ext_width(dn)
    grid = (N // BM,)
    ewp = whe.shape[1]
    in_specs = [
        pl.BlockSpec((BM, 1), lambda i: (i, 0)),
        pl.BlockSpec((1, N), lambda i: (0, 0)),
        pl.BlockSpec((BM, N), lambda i: (i, 0)),
        pl.BlockSpec((N, ewp), lambda i: (0, 0)),
        pl.BlockSpec((W.shape[0], dn), lambda i: (0, 0)),
        pl.BlockSpec((dn, 1), lambda i: (0, 0)),
        pl.BlockSpec((dn, 1), lambda i: (0, 0)),
    ]
    out_specs = [
        pl.BlockSpec((BM, ew), lambda i: (i, 0)),
        pl.BlockSpec((BM, 1), lambda i: (i, 0)),
        pl.BlockSpec((BM, 1), lambda i: (i, 0)),
    ]
    out_shape = [
        jax.ShapeDtypeStruct((N, ew), jnp.bfloat16),
        jax.ShapeDtypeStruct((N, 1), jnp.float32),
        jax.ShapeDtypeStruct((N, 1), jnp.float32),
    ]
    if emit_h:
        out_specs = [pl.BlockSpec((BM, d), lambda i: (i, 0))] + out_specs
        out_shape = [jax.ShapeDtypeStruct((N, d), jnp.float32)] + out_shape
    return pl.pallas_call(
        functools.partial(_att_next_kernel, d=d, emit_h=emit_h),
        grid=grid,
        in_specs=in_specs,
        out_specs=out_specs,
        out_shape=out_shape,
    )(f1, f2.T, mask8, whe, W, a[:dn].reshape(dn, 1), a[dn:].reshape(dn, 1))


BS = 256


@jax.jit
def _run(x, adj, W_e0, a_e0, W_e1, a_e1, W_d0, a_d0, W_d1, a_d1):
    grid = (N // BM,)
    d0 = W_e0.shape[1]
    ew0 = _ext_width(d0)
    whe, f1, f2 = pl.pallas_call(
        _setup_kernel,
        grid=(N // BS,),
        in_specs=[
            pl.BlockSpec((BS, x.shape[1]), lambda i: (i, 0)),
            pl.BlockSpec((x.shape[1], d0), lambda i: (0, 0)),
            pl.BlockSpec((d0, 1), lambda i: (0, 0)),
            pl.BlockSpec((d0, 1), lambda i: (0, 0)),
        ],
        out_specs=[
            pl.BlockSpec((BS, ew0), lambda i: (i, 0)),
            pl.BlockSpec((BS, 1), lambda i: (i, 0)),
            pl.BlockSpec((BS, 1), lambda i: (i, 0)),
        ],
        out_shape=[
            jax.ShapeDtypeStruct((N, ew0), jnp.bfloat16),
            jax.ShapeDtypeStruct((N, 1), jnp.float32),
            jax.ShapeDtypeStruct((N, 1), jnp.float32),
        ],
    )(x, W_e0, a_e0[:d0].reshape(d0, 1), a_e0[d0:].reshape(d0, 1))

    dn1 = W_e1.shape[1]
    ew1 = _ext_width(dn1)
    mask8, whe, f1, f2 = pl.pallas_call(
        functools.partial(_att_first_kernel, d=d0),
        grid=(N // BM,),
        in_specs=[
            pl.BlockSpec((BM, 1), lambda i: (i, 0)),
            pl.BlockSpec((1, N), lambda i: (0, 0)),
            pl.BlockSpec((BM, N), lambda i: (i, 0)),
            pl.BlockSpec((N, ew0), lambda i: (0, 0)),
            pl.BlockSpec((W_e1.shape[0], dn1), lambda i: (0, 0)),
            pl.BlockSpec((dn1, 1), lambda i: (0, 0)),
            pl.BlockSpec((dn1, 1), lambda i: (0, 0)),
        ],
        out_specs=[
            pl.BlockSpec((BM, N), lambda i: (i, 0)),
            pl.BlockSpec((BM, ew1), lambda i: (i, 0)),
            pl.BlockSpec((BM, 1), lambda i: (i, 0)),
            pl.BlockSpec((BM, 1), lambda i: (i, 0)),
        ],
        out_shape=[
            jax.ShapeDtypeStruct((N, N), jnp.float8_e4m3fn),
            jax.ShapeDtypeStruct((N, ew1), jnp.bfloat16),
            jax.ShapeDtypeStruct((N, 1), jnp.float32),
            jax.ShapeDtypeStruct((N, 1), jnp.float32),
        ],
    )(f1, f2.T, adj, whe, W_e1,
      a_e1[:dn1].reshape(dn1, 1), a_e1[dn1:].reshape(dn1, 1))
    d1 = W_e1.shape[1]
    z, whe, f1, f2 = _att_next(f1, f2, mask8, whe, W_d0, a_d0, d1, emit_h=True)
    d2 = W_d0.shape[1]
    whe, f1, f2 = _att_next(f1, f2, mask8, whe, W_d1, a_d1, d2)
    d3 = W_d1.shape[1]
    x_hat = pl.pallas_call(
        functools.partial(_att_last_kernel, d=d3),
        grid=grid,
        in_specs=[
            pl.BlockSpec((BM, 1), lambda i: (i, 0)),
            pl.BlockSpec((1, N), lambda i: (0, 0)),
            pl.BlockSpec((BM, N), lambda i: (i, 0)),
            pl.BlockSpec((N, whe.shape[1]), lambda i: (0, 0)),
        ],
        out_specs=pl.BlockSpec((BM, d3), lambda i: (i, 0)),
        out_shape=jax.ShapeDtypeStruct((N, d3), jnp.float32),
    )(f1, f2.T, mask8, whe)
    return z, x_hat


def kernel(x, adj, W_e0, a_e0, W_e1, a_e1, W_d0, a_d0, W_d1, a_d1):
    return _run(x, adj, W_e0, a_e0, W_e1, a_e1, W_d0, a_d0, W_d1, a_d1)
